# baseline (speedup 1.0000x reference)
import functools

import jax
import jax.numpy as jnp
from jax import lax
from jax.experimental import pallas as pl
from jax.experimental.pallas import tpu as pltpu
from jax.experimental.pallas import tpu_sc as plsc

N = 10000
D = 256
E = 160000
ROWS = 400

NW = 32
NPW = 320
NPAD = NW * NPW
DW = D // 2
C = 1600
NCH = E // C
K = 128
R1 = 3
R2 = 5
CAP = 8192
ECAP = 168448
L = 16


def _proj_body(x_ref, wt_ref, b_ref, o_ref):
    acc = jnp.dot(x_ref[...].astype(jnp.bfloat16), wt_ref[...],
                  preferred_element_type=jnp.float32)
    o_ref[...] = jnp.maximum(acc + b_ref[...], 0.0).astype(jnp.bfloat16)


def _proj(x, wt, b):
    grid = (x.shape[0] // ROWS,)
    return pl.pallas_call(
        _proj_body,
        grid=grid,
        in_specs=[
            pl.BlockSpec((ROWS, D), lambda i: (i, 0)),
            pl.BlockSpec((D, D), lambda i: (0, 0)),
            pl.BlockSpec((1, D), lambda i: (0, 0)),
        ],
        out_specs=pl.BlockSpec((ROWS, D), lambda i: (i, 0)),
        out_shape=jax.ShapeDtypeStruct((x.shape[0], D), jnp.bfloat16),
    )(x, wt.astype(jnp.bfloat16), b)


def _out_body(aggr_ref, wlt_ref, bl_ref, x_ref, wrt_ref, o_ref, *, do_relu):
    acc = jnp.dot(aggr_ref[...].astype(jnp.bfloat16), wlt_ref[...],
                  preferred_element_type=jnp.float32)
    acc = acc + bl_ref[...]
    acc = acc + jnp.dot(x_ref[...].astype(jnp.bfloat16), wrt_ref[...],
                  preferred_element_type=jnp.float32)
    nrm = jnp.sqrt(jnp.sum(acc * acc, axis=-1, keepdims=True))
    res = acc / jnp.maximum(nrm, 1e-12)
    if do_relu:
        res = jnp.maximum(res, 0.0)
    o_ref[...] = res


def _out(aggr, wlt, bl, x, wrt, do_relu):
    grid = (x.shape[0] // ROWS,)
    return pl.pallas_call(
        functools.partial(_out_body, do_relu=do_relu),
        grid=grid,
        in_specs=[
            pl.BlockSpec((ROWS, D), lambda i: (i, 0)),
            pl.BlockSpec((D, D), lambda i: (0, 0)),
            pl.BlockSpec((1, D), lambda i: (0, 0)),
            pl.BlockSpec((ROWS, D), lambda i: (i, 0)),
            pl.BlockSpec((D, D), lambda i: (0, 0)),
        ],
        out_specs=pl.BlockSpec((ROWS, D), lambda i: (i, 0)),
        out_shape=jax.ShapeDtypeStruct((x.shape[0], D), jnp.float32),
    )(aggr, wlt.astype(jnp.bfloat16), bl, x, wrt.astype(jnp.bfloat16))


_GATHER_DNUMS = lax.GatherDimensionNumbers(
    offset_dims=(), collapsed_slice_dims=(0,), start_index_map=(0,))


def _lane_splat(v, kk):
    idx = jnp.full((L, 1), kk, jnp.int32)
    return lax.gather(v, idx, _GATHER_DNUMS, (1,),
                      mode=lax.GatherScatterMode.PROMISE_IN_BOUNDS)


def _zero_aggr(aggr_u, izeros):
    def _z(i, _):
        for w in range(DW // L):
            aggr_u[i, pl.ds(w * L, L)] = izeros
        return 0
    lax.fori_loop(0, NPW, _z, 0)


def _update_block(aggr_u, rbuf, ldbuf, lbase, base, count, iota):
    def _edge(k, _):
        ldg = ldbuf[pl.ds(lbase + (k // L) * L, L)]
        lds = _lane_splat(ldg, k % L)
        valid = jnp.broadcast_to(base + k < count, (L,))
        for w in range(DW // L):
            wvec = iota + w * L
            cur = plsc.bitcast(plsc.load_gather(aggr_u, [lds, wvec]),
                               jnp.bfloat16)
            msg = plsc.bitcast(rbuf[k, pl.ds(w * L, L)], jnp.bfloat16)
            mx = plsc.bitcast(jnp.maximum(cur, msg), jnp.int32)
            plsc.store_scatter(aggr_u, [lds, wvec], mx, mask=valid)
        return 0
    lax.fori_loop(0, K, _edge, 0)


def _seg1_body(xp_hbm, src_hbm, dst_hbm,
               out_hbm, sl_hbm, ll_hbm, cnt_hbm,
               sbufA, dbufA, sbufB, dbufB, src_c, ld_c,
               rows0, rows1, rows2, aggr_u, zsent, lsent, stage,
               semA, semB, semG0, semG1, semG2):
    iota = lax.iota(jnp.int32, L)
    izeros = jnp.zeros((L,), jnp.int32)
    wid = lax.axis_index("s") * 2 + lax.axis_index("c")
    lo = wid * NPW
    lbase0 = pl.multiple_of(wid * ECAP, 8)
    rows = (rows0, rows1, rows2)
    semsG = (semG0, semG1, semG2)

    _zero_aggr(aggr_u, izeros)

    def _z_idx(i, _):
        src_c[pl.ds(i * L, L)] = izeros
        ld_c[pl.ds(i * L, L)] = izeros
        return 0
    lax.fori_loop(0, (CAP + L) // L, _z_idx, 0)

    def _z_sent(i, _):
        zsent[pl.ds(i * L, L)] = izeros
        lsent[pl.ds(i * L, L)] = izeros + NPW
        return 0
    lax.fori_loop(0, K // L, _z_sent, 0)

    def _fire_edges(c, sbuf, dbuf, sem):
        pltpu.async_copy(src_hbm.at[pl.ds(c * C, C)], sbuf, sem)
        pltpu.async_copy(dst_hbm.at[pl.ds(c * C, C)], dbuf, sem)

    def _wait_edges(c, sbuf, dbuf, sem):
        pltpu.make_async_copy(src_hbm.at[pl.ds(c * C, C)], sbuf, sem).wait()
        pltpu.make_async_copy(dst_hbm.at[pl.ds(c * C, C)], dbuf, sem).wait()

    def _scan(sbuf, dbuf, ptr_vec):
        def _it(i, ptr):
            for u in range(4):
                ii = i * 4 + u
                d = dbuf[pl.ds(ii * L, L)]
                s = sbuf[pl.ds(ii * L, L)]
                ld = d - lo
                m = (ld >= 0) & (ld < NPW)
                pos = ptr + plsc.cumsum(jnp.where(m, 1, 0)) - 1
                plsc.store_scatter(src_c, [pos], s, mask=m)
                plsc.store_scatter(ld_c, [pos], ld, mask=m)
                ptr = _lane_splat(pos, L - 1) + 1
            return ptr
        return lax.fori_loop(0, C // L // 4, _it, ptr_vec)

    def _fire_rows(b, r):
        pltpu.async_copy(xp_hbm.at[src_c.at[pl.ds(b * K, K)]],
                         rows[r], semsG[r])

    def _wait_rows(r):
        pltpu.make_async_copy(xp_hbm.at[src_c.at[pl.ds(0, K)]],
                              rows[r], semsG[r]).wait()

    def _drain(count, fired, upd):
        nb = (count + (K - 1)) // K

        def _blk(b, f):
            for _ in range(2):
                canf = (f < nb) & (f - b < R1)
                for r in range(R1):
                    @pl.when(canf & (f % R1 == r))
                    def _():
                        _fire_rows(f, r)
                f = jnp.where(canf, f + 1, f)
            for r in range(R1):
                @pl.when(b % R1 == r)
                def _():
                    _wait_rows(r)
                    _update_block(aggr_u, rows[r], ld_c,
                                  b * K, b * K, count, iota)
            return f
        lax.fori_loop(upd, nb, _blk, fired)

    def _pump(ptr_vec, fired, upd):
        nfull = jnp.max(ptr_vec) // K
        for _ in range(2):
            canf = (fired < nfull) & (fired < upd + R1)
            for r in range(R1):
                @pl.when(canf & (fired % R1 == r))
                def _():
                    _fire_rows(fired, r)
            fired = jnp.where(canf, fired + 1, fired)
        for _ in range(2):
            canu = upd < fired - 1
            for r in range(R1):
                @pl.when(canu & (upd % R1 == r))
                def _():
                    _wait_rows(r)
                    _update_block(aggr_u, rows[r], ld_c,
                                  upd * K, upd * K, (upd + 1) * K, iota)
            upd = jnp.where(canu, upd + 1, upd)
        return fired, upd

    def _flush(ptr_vec, off, fired, upd, force):
        cnt = jnp.max(ptr_vec)
        if force:
            do = cnt > 0
        else:
            do = cnt > CAP - C

        off8 = pl.multiple_of(off, 8)

        @pl.when(do)
        def _():
            plsc.store_scatter(ld_c, [ptr_vec + iota], izeros + NPW,
                               mask=iota < 8)
            pltpu.sync_copy(src_c.at[pl.ds(0, CAP)],
                            sl_hbm.at[pl.ds(lbase0 + off8, CAP)])
            pltpu.sync_copy(ld_c.at[pl.ds(0, CAP)],
                            ll_hbm.at[pl.ds(lbase0 + off8, CAP)])
            _drain(cnt, fired, upd)
        cnt8 = ((cnt + 7) // 8) * 8
        zs = jnp.int32(0)
        return (jnp.where(do, izeros, ptr_vec),
                jnp.where(do, off + cnt8, off),
                jnp.where(do, zs, fired),
                jnp.where(do, zs, upd))

    _fire_edges(0, sbufA, dbufA, semA)

    def _pair(p, carry):
        ptr_vec, off, fired, upd = carry
        c0 = 2 * p
        _wait_edges(c0, sbufA, dbufA, semA)
        _fire_edges(c0 + 1, sbufB, dbufB, semB)
        ptr_vec = _scan(sbufA, dbufA, ptr_vec)
        fired, upd = _pump(ptr_vec, fired, upd)
        ptr_vec, off, fired, upd = _flush(ptr_vec, off, fired, upd, False)
        _wait_edges(c0 + 1, sbufB, dbufB, semB)

        @pl.when(c0 + 2 < NCH)
        def _():
            _fire_edges(c0 + 2, sbufA, dbufA, semA)
        ptr_vec = _scan(sbufB, dbufB, ptr_vec)
        fired, upd = _pump(ptr_vec, fired, upd)
        ptr_vec, off, fired, upd = _flush(ptr_vec, off, fired, upd, False)
        return (ptr_vec, off, fired, upd)

    ptr_vec, off, fired, upd = lax.fori_loop(
        0, NCH // 2, _pair,
        (izeros, jnp.int32(0), jnp.int32(0), jnp.int32(0)))
    ptr_vec, off, fired, upd = _flush(ptr_vec, off, fired, upd, True)

    off8 = pl.multiple_of(off, 8)
    pltpu.sync_copy(zsent, sl_hbm.at[pl.ds(lbase0 + off8, K)])
    pltpu.sync_copy(lsent, ll_hbm.at[pl.ds(lbase0 + off8, K)])
    stage[pl.ds(0, L)] = izeros + off
    pltpu.sync_copy(stage, cnt_hbm.at[pl.ds(wid * L, L)])

    pltpu.sync_copy(aggr_u.at[pl.ds(0, NPW)], out_hbm.at[pl.ds(lo, NPW)])


_seg1_kernel = functools.partial(
    pl.kernel,
    out_type=(
        jax.ShapeDtypeStruct((NPAD, DW), jnp.int32),
        jax.ShapeDtypeStruct((NW * ECAP,), jnp.int32),
        jax.ShapeDtypeStruct((NW * ECAP,), jnp.int32),
        jax.ShapeDtypeStruct((NW * L,), jnp.int32),
    ),
    mesh=plsc.VectorSubcoreMesh(core_axis_name="c", subcore_axis_name="s"),
    compiler_params=pltpu.CompilerParams(needs_layout_passes=False),
    scratch_types=[
        pltpu.VMEM((C,), jnp.int32),
        pltpu.VMEM((C,), jnp.int32),
        pltpu.VMEM((C,), jnp.int32),
        pltpu.VMEM((C,), jnp.int32),
        pltpu.VMEM((CAP + L,), jnp.int32),
        pltpu.VMEM((CAP + L,), jnp.int32),
        pltpu.VMEM((K, DW), jnp.int32),
        pltpu.VMEM((K, DW), jnp.int32),
        pltpu.VMEM((K, DW), jnp.int32),
        pltpu.VMEM((NPW + 1, DW), jnp.int32),
        pltpu.VMEM((K,), jnp.int32),
        pltpu.VMEM((K,), jnp.int32),
        pltpu.VMEM((L,), jnp.int32),
        pltpu.SemaphoreType.DMA,
        pltpu.SemaphoreType.DMA,
        pltpu.SemaphoreType.DMA,
        pltpu.SemaphoreType.DMA,
        pltpu.SemaphoreType.DMA,
    ],
)(_seg1_body)


def _seg2_body(xp_hbm, sl_hbm, ll_hbm, cnt_hbm, out_hbm,
               sidx0, sidx1, sidx2, sidx3, sidx4,
               lidx0, lidx1, lidx2, lidx3, lidx4,
               rows0, rows1, rows2, rows3, rows4, aggr_u, cbuf,
               semL0, semL1, semL2, semL3, semL4,
               semG0, semG1, semG2, semG3, semG4):
    iota = lax.iota(jnp.int32, L)
    izeros = jnp.zeros((L,), jnp.int32)
    wid = lax.axis_index("s") * 2 + lax.axis_index("c")
    lo = wid * NPW
    lbase0 = pl.multiple_of(wid * ECAP, 8)
    sidx = (sidx0, sidx1, sidx2, sidx3, sidx4)
    lidx = (lidx0, lidx1, lidx2, lidx3, lidx4)
    rows = (rows0, rows1, rows2, rows3, rows4)
    semsL = (semL0, semL1, semL2, semL3, semL4)
    semsG = (semG0, semG1, semG2, semG3, semG4)

    _zero_aggr(aggr_u, izeros)

    pltpu.sync_copy(cnt_hbm.at[pl.ds(wid * L, L)], cbuf)
    total = jnp.max(cbuf[pl.ds(0, L)])
    nb = (total + (K - 1)) // K

    def _fire_list(b, r):
        o = pl.multiple_of(lbase0 + b * K, 8)
        pltpu.async_copy(sl_hbm.at[pl.ds(o, K)], sidx[r], semsL[r])
        pltpu.async_copy(ll_hbm.at[pl.ds(o, K)], lidx[r], semsL[r])

    def _wait_list(r):
        pltpu.make_async_copy(sl_hbm.at[pl.ds(lbase0, K)],
                              sidx[r], semsL[r]).wait()
        pltpu.make_async_copy(ll_hbm.at[pl.ds(lbase0, K)],
                              lidx[r], semsL[r]).wait()

    def _fire_rows(r):
        pltpu.async_copy(xp_hbm.at[sidx[r]], rows[r], semsG[r])

    def _wait_rows(r):
        pltpu.make_async_copy(xp_hbm.at[sidx[r]], rows[r], semsG[r]).wait()

    for j in range(4):
        @pl.when(j < nb)
        def _():
            _fire_list(jnp.int32(j), j)
    for j in range(2):
        @pl.when(j < nb)
        def _():
            _wait_list(j)
            _fire_rows(j)

    def _step(ts, _):
        for r in range(R2):
            t = ts * R2 + r

            @pl.when(t + 4 < nb)
            def _():
                _fire_list(t + 4, (r + 4) % R2)

            @pl.when(t + 2 < nb)
            def _():
                _wait_list((r + 2) % R2)
                _fire_rows((r + 2) % R2)

            @pl.when(t < nb)
            def _():
                _wait_rows(r)
                _update_block(aggr_u, rows[r], lidx[r], 0, t * K, total, iota)
        return 0
    lax.fori_loop(0, (nb + (R2 - 1)) // R2, _step, 0)

    pltpu.sync_copy(aggr_u.at[pl.ds(0, NPW)], out_hbm.at[pl.ds(lo, NPW)])


_seg2_kernel = functools.partial(
    pl.kernel,
    out_type=jax.ShapeDtypeStruct((NPAD, DW), jnp.int32),
    mesh=plsc.VectorSubcoreMesh(core_axis_name="c", subcore_axis_name="s"),
    compiler_params=pltpu.CompilerParams(needs_layout_passes=False),
    scratch_types=(
        [pltpu.VMEM((K,), jnp.int32) for _ in range(5)]
        + [pltpu.VMEM((K,), jnp.int32) for _ in range(5)]
        + [pltpu.VMEM((K, DW), jnp.int32) for _ in range(5)]
        + [pltpu.VMEM((NPW + 1, DW), jnp.int32)]
        + [pltpu.VMEM((L,), jnp.int32)]
        + [pltpu.SemaphoreType.DMA for _ in range(10)]
    ),
)(_seg2_body)


def _pack(xp_bf16):
    return lax.bitcast_convert_type(xp_bf16.reshape(N, DW, 2), jnp.int32)


def _unpack(aggr_u):
    aggr = lax.bitcast_convert_type(aggr_u, jnp.bfloat16)
    return aggr.reshape(NPAD, D)[:N].astype(jnp.float32)


@jax.jit
def kernel(x, edge_index, Wp1, bp1, Wl1, bl1, Wr1, Wp2, bp2, Wl2, bl2, Wr2):
    src = edge_index[0]
    dst = edge_index[1]

    xp1 = _proj(x, Wp1.T, bp1.reshape(1, D))
    aggr1_u, sl, ll, cnt = _seg1_kernel(_pack(xp1), src, dst)
    h = _out(_unpack(aggr1_u), Wl1.T, bl1.reshape(1, D), x, Wr1.T, True)

    xp2 = _proj(h, Wp2.T, bp2.reshape(1, D))
    aggr2_u = _seg2_kernel(_pack(xp2), sl, ll, cnt)
    return _out(_unpack(aggr2_u), Wl2.T, bl2.reshape(1, D), h, Wr2.T, False)


# speedup vs baseline: 1.0343x; 1.0343x over previous
import functools

import jax
import jax.numpy as jnp
from jax import lax
from jax.experimental import pallas as pl
from jax.experimental.pallas import tpu as pltpu
from jax.experimental.pallas import tpu_sc as plsc

N = 10000
D = 256
E = 160000
ROWS = 400

NW = 32
NPW = 320
NPAD = NW * NPW
DW = D // 2
C = 1600
NCH = E // C
K = 128
R1 = 3
R2 = 5
CAP = 8192
ECAP = 168448
L = 16


def _proj_body(x_ref, wt_ref, b_ref, o_ref):
    acc = jnp.dot(x_ref[...].astype(jnp.bfloat16), wt_ref[...],
                  preferred_element_type=jnp.float32)
    o_ref[...] = jnp.maximum(acc + b_ref[...], 0.0).astype(jnp.bfloat16)


def _proj(x, wt, b):
    grid = (x.shape[0] // ROWS,)
    return pl.pallas_call(
        _proj_body,
        grid=grid,
        in_specs=[
            pl.BlockSpec((ROWS, D), lambda i: (i, 0)),
            pl.BlockSpec((D, D), lambda i: (0, 0)),
            pl.BlockSpec((1, D), lambda i: (0, 0)),
        ],
        out_specs=pl.BlockSpec((ROWS, D), lambda i: (i, 0)),
        out_shape=jax.ShapeDtypeStruct((x.shape[0], D), jnp.bfloat16),
    )(x, wt.astype(jnp.bfloat16), b)


def _out_body(aggr_ref, wlt_ref, bl_ref, x_ref, wrt_ref, o_ref, *, do_relu):
    acc = jnp.dot(aggr_ref[...].astype(jnp.bfloat16), wlt_ref[...],
                  preferred_element_type=jnp.float32)
    acc = acc + bl_ref[...]
    acc = acc + jnp.dot(x_ref[...].astype(jnp.bfloat16), wrt_ref[...],
                  preferred_element_type=jnp.float32)
    nrm = jnp.sqrt(jnp.sum(acc * acc, axis=-1, keepdims=True))
    res = acc / jnp.maximum(nrm, 1e-12)
    if do_relu:
        res = jnp.maximum(res, 0.0)
    o_ref[...] = res


def _out(aggr, wlt, bl, x, wrt, do_relu):
    grid = (x.shape[0] // ROWS,)
    return pl.pallas_call(
        functools.partial(_out_body, do_relu=do_relu),
        grid=grid,
        in_specs=[
            pl.BlockSpec((ROWS, D), lambda i: (i, 0)),
            pl.BlockSpec((D, D), lambda i: (0, 0)),
            pl.BlockSpec((1, D), lambda i: (0, 0)),
            pl.BlockSpec((ROWS, D), lambda i: (i, 0)),
            pl.BlockSpec((D, D), lambda i: (0, 0)),
        ],
        out_specs=pl.BlockSpec((ROWS, D), lambda i: (i, 0)),
        out_shape=jax.ShapeDtypeStruct((x.shape[0], D), jnp.float32),
    )(aggr, wlt.astype(jnp.bfloat16), bl, x, wrt.astype(jnp.bfloat16))


_GATHER_DNUMS = lax.GatherDimensionNumbers(
    offset_dims=(), collapsed_slice_dims=(0,), start_index_map=(0,))


def _lane_splat(v, kk):
    idx = jnp.full((L, 1), kk, jnp.int32)
    return lax.gather(v, idx, _GATHER_DNUMS, (1,),
                      mode=lax.GatherScatterMode.PROMISE_IN_BOUNDS)


def _zero_aggr(aggr_u, izeros):
    def _z(i, _):
        for w in range(DW // L):
            aggr_u[i, pl.ds(w * L, L)] = izeros
        return 0
    lax.fori_loop(0, NPW, _z, 0)


def _update_block(aggr_u, rbuf, ldbuf, lbase, base, count, iota):
    def _edge(k, _):
        ldg = ldbuf[pl.ds(lbase + (k // L) * L, L)]
        lds = _lane_splat(ldg, k % L)
        valid = jnp.broadcast_to(base + k < count, (L,))
        for w in range(DW // L):
            wvec = iota + w * L
            cur = plsc.bitcast(plsc.load_gather(aggr_u, [lds, wvec]),
                               jnp.bfloat16)
            msg = plsc.bitcast(rbuf[k, pl.ds(w * L, L)], jnp.bfloat16)
            mx = plsc.bitcast(jnp.maximum(cur, msg), jnp.int32)
            plsc.store_scatter(aggr_u, [lds, wvec], mx, mask=valid)
        return 0
    lax.fori_loop(0, K, _edge, 0)


def _seg1_body(xp_hbm, src_hbm, dst_hbm,
               out_hbm, sl_hbm, ll_hbm, cnt_hbm,
               sbufA, dbufA, sbufB, dbufB, src_c, ld_c,
               rows0, rows1, rows2, aggr_u, zsent, lsent, stage,
               semA, semB, semG0, semG1, semG2):
    iota = lax.iota(jnp.int32, L)
    izeros = jnp.zeros((L,), jnp.int32)
    wid = lax.axis_index("s") * 2 + lax.axis_index("c")
    lo = wid * NPW
    lbase0 = pl.multiple_of(wid * ECAP, 8)
    rows = (rows0, rows1, rows2)
    semsG = (semG0, semG1, semG2)

    _zero_aggr(aggr_u, izeros)

    def _z_idx(i, _):
        src_c[pl.ds(i * L, L)] = izeros
        ld_c[pl.ds(i * L, L)] = izeros
        return 0
    lax.fori_loop(0, (CAP + L) // L, _z_idx, 0)

    def _z_sent(i, _):
        zsent[pl.ds(i * L, L)] = izeros
        lsent[pl.ds(i * L, L)] = izeros + NPW
        return 0
    lax.fori_loop(0, K // L, _z_sent, 0)

    def _fire_edges(c, sbuf, dbuf, sem):
        pltpu.async_copy(src_hbm.at[pl.ds(c * C, C)], sbuf, sem)
        pltpu.async_copy(dst_hbm.at[pl.ds(c * C, C)], dbuf, sem)

    def _wait_edges(c, sbuf, dbuf, sem):
        pltpu.make_async_copy(src_hbm.at[pl.ds(c * C, C)], sbuf, sem).wait()
        pltpu.make_async_copy(dst_hbm.at[pl.ds(c * C, C)], dbuf, sem).wait()

    def _scan(sbuf, dbuf, ptr_vec):
        def _it(i, ptr):
            for u in range(4):
                ii = i * 4 + u
                d = dbuf[pl.ds(ii * L, L)]
                s = sbuf[pl.ds(ii * L, L)]
                ld = d - lo
                m = (ld >= 0) & (ld < NPW)
                pos = ptr + plsc.cumsum(jnp.where(m, 1, 0)) - 1
                plsc.store_scatter(src_c, [pos], s, mask=m)
                plsc.store_scatter(ld_c, [pos], ld, mask=m)
                ptr = _lane_splat(pos, L - 1) + 1
            return ptr
        return lax.fori_loop(0, C // L // 4, _it, ptr_vec)

    def _fire_rows(b, r):
        pltpu.async_copy(xp_hbm.at[src_c.at[pl.ds(b * K, K)]],
                         rows[r], semsG[r])

    def _wait_rows(r):
        pltpu.make_async_copy(xp_hbm.at[src_c.at[pl.ds(0, K)]],
                              rows[r], semsG[r]).wait()

    def _drain(count):
        nb = (count + (K - 1)) // K
        for r in range(R1):
            @pl.when(r < nb)
            def _():
                _fire_rows(jnp.int32(r), r)

        def _super(sb, _):
            for r in range(R1):
                b = sb * R1 + r

                @pl.when(b < nb)
                def _():
                    _wait_rows(r)
                    _update_block(aggr_u, rows[r], ld_c,
                                  b * K, b * K, count, iota)

                    @pl.when(b + R1 < nb)
                    def _():
                        _fire_rows(b + R1, r)
            return 0
        lax.fori_loop(0, (nb + (R1 - 1)) // R1, _super, 0)

    def _flush(ptr_vec, off, force):
        cnt = jnp.max(ptr_vec)
        if force:
            do = cnt > 0
        else:
            do = cnt > CAP - C

        off8 = pl.multiple_of(off, 8)

        @pl.when(do)
        def _():
            plsc.store_scatter(ld_c, [ptr_vec + iota], izeros + NPW,
                               mask=iota < 8)
            pltpu.sync_copy(src_c.at[pl.ds(0, CAP)],
                            sl_hbm.at[pl.ds(lbase0 + off8, CAP)])
            pltpu.sync_copy(ld_c.at[pl.ds(0, CAP)],
                            ll_hbm.at[pl.ds(lbase0 + off8, CAP)])
            _drain(cnt)
        cnt8 = ((cnt + 7) // 8) * 8
        return (jnp.where(do, izeros, ptr_vec),
                jnp.where(do, off + cnt8, off))

    _fire_edges(0, sbufA, dbufA, semA)

    def _pair(p, carry):
        ptr_vec, off = carry
        c0 = 2 * p
        _wait_edges(c0, sbufA, dbufA, semA)
        _fire_edges(c0 + 1, sbufB, dbufB, semB)
        ptr_vec, off = _flush(_scan(sbufA, dbufA, ptr_vec), off, False)
        _wait_edges(c0 + 1, sbufB, dbufB, semB)

        @pl.when(c0 + 2 < NCH)
        def _():
            _fire_edges(c0 + 2, sbufA, dbufA, semA)
        ptr_vec, off = _flush(_scan(sbufB, dbufB, ptr_vec), off, False)
        return (ptr_vec, off)

    ptr_vec, off = lax.fori_loop(0, NCH // 2, _pair, (izeros, jnp.int32(0)))
    ptr_vec, off = _flush(ptr_vec, off, True)

    off8 = pl.multiple_of(off, 8)
    pltpu.sync_copy(zsent, sl_hbm.at[pl.ds(lbase0 + off8, K)])
    pltpu.sync_copy(lsent, ll_hbm.at[pl.ds(lbase0 + off8, K)])
    stage[pl.ds(0, L)] = izeros + off
    pltpu.sync_copy(stage, cnt_hbm.at[pl.ds(wid * L, L)])

    pltpu.sync_copy(aggr_u.at[pl.ds(0, NPW)], out_hbm.at[pl.ds(lo, NPW)])


_seg1_kernel = functools.partial(
    pl.kernel,
    out_type=(
        jax.ShapeDtypeStruct((NPAD, DW), jnp.int32),
        jax.ShapeDtypeStruct((NW * ECAP,), jnp.int32),
        jax.ShapeDtypeStruct((NW * ECAP,), jnp.int32),
        jax.ShapeDtypeStruct((NW * L,), jnp.int32),
    ),
    mesh=plsc.VectorSubcoreMesh(core_axis_name="c", subcore_axis_name="s"),
    compiler_params=pltpu.CompilerParams(needs_layout_passes=False),
    scratch_types=[
        pltpu.VMEM((C,), jnp.int32),
        pltpu.VMEM((C,), jnp.int32),
        pltpu.VMEM((C,), jnp.int32),
        pltpu.VMEM((C,), jnp.int32),
        pltpu.VMEM((CAP + L,), jnp.int32),
        pltpu.VMEM((CAP + L,), jnp.int32),
        pltpu.VMEM((K, DW), jnp.int32),
        pltpu.VMEM((K, DW), jnp.int32),
        pltpu.VMEM((K, DW), jnp.int32),
        pltpu.VMEM((NPW + 1, DW), jnp.int32),
        pltpu.VMEM((K,), jnp.int32),
        pltpu.VMEM((K,), jnp.int32),
        pltpu.VMEM((L,), jnp.int32),
        pltpu.SemaphoreType.DMA,
        pltpu.SemaphoreType.DMA,
        pltpu.SemaphoreType.DMA,
        pltpu.SemaphoreType.DMA,
        pltpu.SemaphoreType.DMA,
    ],
)(_seg1_body)


def _seg2_body(xp_hbm, sl_hbm, ll_hbm, cnt_hbm, out_hbm,
               sidx0, sidx1, sidx2, sidx3, sidx4,
               lidx0, lidx1, lidx2, lidx3, lidx4,
               rows0, rows1, rows2, rows3, rows4, aggr_u, cbuf,
               semL0, semL1, semL2, semL3, semL4,
               semG0, semG1, semG2, semG3, semG4):
    iota = lax.iota(jnp.int32, L)
    izeros = jnp.zeros((L,), jnp.int32)
    wid = lax.axis_index("s") * 2 + lax.axis_index("c")
    lo = wid * NPW
    lbase0 = pl.multiple_of(wid * ECAP, 8)
    sidx = (sidx0, sidx1, sidx2, sidx3, sidx4)
    lidx = (lidx0, lidx1, lidx2, lidx3, lidx4)
    rows = (rows0, rows1, rows2, rows3, rows4)
    semsL = (semL0, semL1, semL2, semL3, semL4)
    semsG = (semG0, semG1, semG2, semG3, semG4)

    _zero_aggr(aggr_u, izeros)

    pltpu.sync_copy(cnt_hbm.at[pl.ds(wid * L, L)], cbuf)
    total = jnp.max(cbuf[pl.ds(0, L)])
    nb = (total + (K - 1)) // K

    def _fire_list(b, r):
        o = pl.multiple_of(lbase0 + b * K, 8)
        pltpu.async_copy(sl_hbm.at[pl.ds(o, K)], sidx[r], semsL[r])
        pltpu.async_copy(ll_hbm.at[pl.ds(o, K)], lidx[r], semsL[r])

    def _wait_list(r):
        pltpu.make_async_copy(sl_hbm.at[pl.ds(lbase0, K)],
                              sidx[r], semsL[r]).wait()
        pltpu.make_async_copy(ll_hbm.at[pl.ds(lbase0, K)],
                              lidx[r], semsL[r]).wait()

    def _fire_rows(r):
        pltpu.async_copy(xp_hbm.at[sidx[r]], rows[r], semsG[r])

    def _wait_rows(r):
        pltpu.make_async_copy(xp_hbm.at[sidx[r]], rows[r], semsG[r]).wait()

    for j in range(4):
        @pl.when(j < nb)
        def _():
            _fire_list(jnp.int32(j), j)
    for j in range(2):
        @pl.when(j < nb)
        def _():
            _wait_list(j)
            _fire_rows(j)

    def _step(ts, _):
        for r in range(R2):
            t = ts * R2 + r

            @pl.when(t + 4 < nb)
            def _():
                _fire_list(t + 4, (r + 4) % R2)

            @pl.when(t + 2 < nb)
            def _():
                _wait_list((r + 2) % R2)
                _fire_rows((r + 2) % R2)

            @pl.when(t < nb)
            def _():
                _wait_rows(r)
                _update_block(aggr_u, rows[r], lidx[r], 0, t * K, total, iota)
        return 0
    lax.fori_loop(0, (nb + (R2 - 1)) // R2, _step, 0)

    pltpu.sync_copy(aggr_u.at[pl.ds(0, NPW)], out_hbm.at[pl.ds(lo, NPW)])


_seg2_kernel = functools.partial(
    pl.kernel,
    out_type=jax.ShapeDtypeStruct((NPAD, DW), jnp.int32),
    mesh=plsc.VectorSubcoreMesh(core_axis_name="c", subcore_axis_name="s"),
    compiler_params=pltpu.CompilerParams(needs_layout_passes=False),
    scratch_types=(
        [pltpu.VMEM((K,), jnp.int32) for _ in range(5)]
        + [pltpu.VMEM((K,), jnp.int32) for _ in range(5)]
        + [pltpu.VMEM((K, DW), jnp.int32) for _ in range(5)]
        + [pltpu.VMEM((NPW + 1, DW), jnp.int32)]
        + [pltpu.VMEM((L,), jnp.int32)]
        + [pltpu.SemaphoreType.DMA for _ in range(10)]
    ),
)(_seg2_body)


def _pack(xp_bf16):
    return lax.bitcast_convert_type(xp_bf16.reshape(N, DW, 2), jnp.int32)


def _unpack(aggr_u):
    aggr = lax.bitcast_convert_type(aggr_u, jnp.bfloat16)
    return aggr.reshape(NPAD, D)[:N].astype(jnp.float32)


@jax.jit
def kernel(x, edge_index, Wp1, bp1, Wl1, bl1, Wr1, Wp2, bp2, Wl2, bl2, Wr2):
    src = edge_index[0]
    dst = edge_index[1]

    xp1 = _proj(x, Wp1.T, bp1.reshape(1, D))
    aggr1_u, sl, ll, cnt = _seg1_kernel(_pack(xp1), src, dst)
    h = _out(_unpack(aggr1_u), Wl1.T, bl1.reshape(1, D), x, Wr1.T, True)

    xp2 = _proj(h, Wp2.T, bp2.reshape(1, D))
    aggr2_u = _seg2_kernel(_pack(xp2), sl, ll, cnt)
    return _out(_unpack(aggr2_u), Wl2.T, bl2.reshape(1, D), h, Wr2.T, False)


# speedup vs baseline: 1.0361x; 1.0017x over previous
import functools

import jax
import jax.numpy as jnp
from jax import lax
from jax.experimental import pallas as pl
from jax.experimental.pallas import tpu as pltpu
from jax.experimental.pallas import tpu_sc as plsc

N = 10000
D = 256
E = 160000
ROWS = 400

NW = 32
NPW = 320
NPAD = NW * NPW
DW = D // 2
C = 1600
NCH = E // C
K = 128
R1 = 3
R2 = 5
CAP = 8192
ECAP = 168448
L = 16


def _proj_body(x_ref, wt_ref, b_ref, o_ref):
    acc = jnp.dot(x_ref[...].astype(jnp.bfloat16), wt_ref[...],
                  preferred_element_type=jnp.float32)
    o_ref[...] = jnp.maximum(acc + b_ref[...], 0.0).astype(jnp.bfloat16)


def _proj(x, wt, b):
    grid = (x.shape[0] // ROWS,)
    return pl.pallas_call(
        _proj_body,
        grid=grid,
        in_specs=[
            pl.BlockSpec((ROWS, D), lambda i: (i, 0)),
            pl.BlockSpec((D, D), lambda i: (0, 0)),
            pl.BlockSpec((1, D), lambda i: (0, 0)),
        ],
        out_specs=pl.BlockSpec((ROWS, D), lambda i: (i, 0)),
        out_shape=jax.ShapeDtypeStruct((x.shape[0], D), jnp.bfloat16),
    )(x, wt.astype(jnp.bfloat16), b)


def _out_body(aggr_ref, wlt_ref, bl_ref, x_ref, wrt_ref, o_ref, *, do_relu):
    acc = jnp.dot(aggr_ref[...].astype(jnp.bfloat16), wlt_ref[...],
                  preferred_element_type=jnp.float32)
    acc = acc + bl_ref[...]
    acc = acc + jnp.dot(x_ref[...].astype(jnp.bfloat16), wrt_ref[...],
                  preferred_element_type=jnp.float32)
    nrm = jnp.sqrt(jnp.sum(acc * acc, axis=-1, keepdims=True))
    res = acc / jnp.maximum(nrm, 1e-12)
    if do_relu:
        res = jnp.maximum(res, 0.0)
    o_ref[...] = res


def _out(aggr, wlt, bl, x, wrt, do_relu):
    grid = (x.shape[0] // ROWS,)
    return pl.pallas_call(
        functools.partial(_out_body, do_relu=do_relu),
        grid=grid,
        in_specs=[
            pl.BlockSpec((ROWS, D), lambda i: (i, 0)),
            pl.BlockSpec((D, D), lambda i: (0, 0)),
            pl.BlockSpec((1, D), lambda i: (0, 0)),
            pl.BlockSpec((ROWS, D), lambda i: (i, 0)),
            pl.BlockSpec((D, D), lambda i: (0, 0)),
        ],
        out_specs=pl.BlockSpec((ROWS, D), lambda i: (i, 0)),
        out_shape=jax.ShapeDtypeStruct((x.shape[0], D), jnp.float32),
    )(aggr, wlt.astype(jnp.bfloat16), bl, x, wrt.astype(jnp.bfloat16))


_GATHER_DNUMS = lax.GatherDimensionNumbers(
    offset_dims=(), collapsed_slice_dims=(0,), start_index_map=(0,))


def _lane_splat(v, kk):
    idx = jnp.full((L, 1), kk, jnp.int32)
    return lax.gather(v, idx, _GATHER_DNUMS, (1,),
                      mode=lax.GatherScatterMode.PROMISE_IN_BOUNDS)


def _zero_aggr(aggr_u, izeros):
    def _z(i, _):
        for w in range(DW // L):
            aggr_u[i, pl.ds(w * L, L)] = izeros
        return 0
    lax.fori_loop(0, NPW, _z, 0)


def _update_block(aggr_u, rbuf, ldbuf, lbase, base, count, iota):
    def _edge(k, _):
        ldg = ldbuf[pl.ds(lbase + (k // L) * L, L)]
        lds = _lane_splat(ldg, k % L)
        valid = jnp.broadcast_to(base + k < count, (L,))
        for w in range(DW // L):
            wvec = iota + w * L
            cur = plsc.bitcast(plsc.load_gather(aggr_u, [lds, wvec]),
                               jnp.bfloat16)
            msg = plsc.bitcast(rbuf[k, pl.ds(w * L, L)], jnp.bfloat16)
            mx = plsc.bitcast(jnp.maximum(cur, msg), jnp.int32)
            plsc.store_scatter(aggr_u, [lds, wvec], mx, mask=valid)
        return 0
    lax.fori_loop(0, K, _edge, 0)


def _seg1_body(xp_hbm, src_hbm, dst_hbm,
               out_hbm, sl_hbm, ll_hbm, cnt_hbm,
               sbufA, dbufA, sbufB, dbufB, src_c, ld_c,
               rows0, rows1, rows2, aggr_u, zsent, lsent, stage,
               semA, semB, semG0, semG1, semG2):
    iota = lax.iota(jnp.int32, L)
    izeros = jnp.zeros((L,), jnp.int32)
    wid = lax.axis_index("s") * 2 + lax.axis_index("c")
    lo = wid * NPW
    lbase0 = pl.multiple_of(wid * ECAP, 8)
    rows = (rows0, rows1, rows2)
    semsG = (semG0, semG1, semG2)

    _zero_aggr(aggr_u, izeros)

    def _z_idx(i, _):
        src_c[pl.ds(i * L, L)] = izeros
        ld_c[pl.ds(i * L, L)] = izeros
        return 0
    lax.fori_loop(0, (CAP + L) // L, _z_idx, 0)

    def _z_sent(i, _):
        zsent[pl.ds(i * L, L)] = izeros
        lsent[pl.ds(i * L, L)] = izeros + NPW
        return 0
    lax.fori_loop(0, K // L, _z_sent, 0)

    def _fire_edges(c, sbuf, dbuf, sem):
        pltpu.async_copy(src_hbm.at[pl.ds(c * C, C)], sbuf, sem)
        pltpu.async_copy(dst_hbm.at[pl.ds(c * C, C)], dbuf, sem)

    def _wait_edges(c, sbuf, dbuf, sem):
        pltpu.make_async_copy(src_hbm.at[pl.ds(c * C, C)], sbuf, sem).wait()
        pltpu.make_async_copy(dst_hbm.at[pl.ds(c * C, C)], dbuf, sem).wait()

    def _scan(sbuf, dbuf, ptr_vec):
        def _it(i, ptr):
            for u in range(10):
                ii = i * 10 + u
                d = dbuf[pl.ds(ii * L, L)]
                s = sbuf[pl.ds(ii * L, L)]
                ld = d - lo
                m = (ld >= 0) & (ld < NPW)
                pos = ptr + plsc.cumsum(jnp.where(m, 1, 0)) - 1
                plsc.store_scatter(src_c, [pos], s, mask=m)
                plsc.store_scatter(ld_c, [pos], ld, mask=m)
                ptr = _lane_splat(pos, L - 1) + 1
            return ptr
        return lax.fori_loop(0, C // L // 10, _it, ptr_vec)

    def _fire_rows(b, r):
        pltpu.async_copy(xp_hbm.at[src_c.at[pl.ds(b * K, K)]],
                         rows[r], semsG[r])

    def _wait_rows(r):
        pltpu.make_async_copy(xp_hbm.at[src_c.at[pl.ds(0, K)]],
                              rows[r], semsG[r]).wait()

    def _drain(count):
        nb = (count + (K - 1)) // K
        for r in range(R1):
            @pl.when(r < nb)
            def _():
                _fire_rows(jnp.int32(r), r)

        def _super(sb, _):
            for r in range(R1):
                b = sb * R1 + r

                @pl.when(b < nb)
                def _():
                    _wait_rows(r)
                    _update_block(aggr_u, rows[r], ld_c,
                                  b * K, b * K, count, iota)

                    @pl.when(b + R1 < nb)
                    def _():
                        _fire_rows(b + R1, r)
            return 0
        lax.fori_loop(0, (nb + (R1 - 1)) // R1, _super, 0)

    def _flush(ptr_vec, off, force):
        cnt = jnp.max(ptr_vec)
        if force:
            do = cnt > 0
        else:
            do = cnt > CAP - C

        off8 = pl.multiple_of(off, 8)

        @pl.when(do)
        def _():
            plsc.store_scatter(ld_c, [ptr_vec + iota], izeros + NPW,
                               mask=iota < 8)
            pltpu.sync_copy(src_c.at[pl.ds(0, CAP)],
                            sl_hbm.at[pl.ds(lbase0 + off8, CAP)])
            pltpu.sync_copy(ld_c.at[pl.ds(0, CAP)],
                            ll_hbm.at[pl.ds(lbase0 + off8, CAP)])
            _drain(cnt)
        cnt8 = ((cnt + 7) // 8) * 8
        return (jnp.where(do, izeros, ptr_vec),
                jnp.where(do, off + cnt8, off))

    _fire_edges(0, sbufA, dbufA, semA)

    def _pair(p, carry):
        ptr_vec, off = carry
        c0 = 2 * p
        _wait_edges(c0, sbufA, dbufA, semA)
        _fire_edges(c0 + 1, sbufB, dbufB, semB)
        ptr_vec, off = _flush(_scan(sbufA, dbufA, ptr_vec), off, False)
        _wait_edges(c0 + 1, sbufB, dbufB, semB)

        @pl.when(c0 + 2 < NCH)
        def _():
            _fire_edges(c0 + 2, sbufA, dbufA, semA)
        ptr_vec, off = _flush(_scan(sbufB, dbufB, ptr_vec), off, False)
        return (ptr_vec, off)

    ptr_vec, off = lax.fori_loop(0, NCH // 2, _pair, (izeros, jnp.int32(0)))
    ptr_vec, off = _flush(ptr_vec, off, True)

    off8 = pl.multiple_of(off, 8)
    pltpu.sync_copy(zsent, sl_hbm.at[pl.ds(lbase0 + off8, K)])
    pltpu.sync_copy(lsent, ll_hbm.at[pl.ds(lbase0 + off8, K)])
    stage[pl.ds(0, L)] = izeros + off
    pltpu.sync_copy(stage, cnt_hbm.at[pl.ds(wid * L, L)])

    pltpu.sync_copy(aggr_u.at[pl.ds(0, NPW)], out_hbm.at[pl.ds(lo, NPW)])


_seg1_kernel = functools.partial(
    pl.kernel,
    out_type=(
        jax.ShapeDtypeStruct((NPAD, DW), jnp.int32),
        jax.ShapeDtypeStruct((NW * ECAP,), jnp.int32),
        jax.ShapeDtypeStruct((NW * ECAP,), jnp.int32),
        jax.ShapeDtypeStruct((NW * L,), jnp.int32),
    ),
    mesh=plsc.VectorSubcoreMesh(core_axis_name="c", subcore_axis_name="s"),
    compiler_params=pltpu.CompilerParams(needs_layout_passes=False),
    scratch_types=[
        pltpu.VMEM((C,), jnp.int32),
        pltpu.VMEM((C,), jnp.int32),
        pltpu.VMEM((C,), jnp.int32),
        pltpu.VMEM((C,), jnp.int32),
        pltpu.VMEM((CAP + L,), jnp.int32),
        pltpu.VMEM((CAP + L,), jnp.int32),
        pltpu.VMEM((K, DW), jnp.int32),
        pltpu.VMEM((K, DW), jnp.int32),
        pltpu.VMEM((K, DW), jnp.int32),
        pltpu.VMEM((NPW + 1, DW), jnp.int32),
        pltpu.VMEM((K,), jnp.int32),
        pltpu.VMEM((K,), jnp.int32),
        pltpu.VMEM((L,), jnp.int32),
        pltpu.SemaphoreType.DMA,
        pltpu.SemaphoreType.DMA,
        pltpu.SemaphoreType.DMA,
        pltpu.SemaphoreType.DMA,
        pltpu.SemaphoreType.DMA,
    ],
)(_seg1_body)


def _seg2_body(xp_hbm, sl_hbm, ll_hbm, cnt_hbm, out_hbm,
               sidx0, sidx1, sidx2, sidx3, sidx4,
               lidx0, lidx1, lidx2, lidx3, lidx4,
               rows0, rows1, rows2, rows3, rows4, aggr_u, cbuf,
               semL0, semL1, semL2, semL3, semL4,
               semG0, semG1, semG2, semG3, semG4):
    iota = lax.iota(jnp.int32, L)
    izeros = jnp.zeros((L,), jnp.int32)
    wid = lax.axis_index("s") * 2 + lax.axis_index("c")
    lo = wid * NPW
    lbase0 = pl.multiple_of(wid * ECAP, 8)
    sidx = (sidx0, sidx1, sidx2, sidx3, sidx4)
    lidx = (lidx0, lidx1, lidx2, lidx3, lidx4)
    rows = (rows0, rows1, rows2, rows3, rows4)
    semsL = (semL0, semL1, semL2, semL3, semL4)
    semsG = (semG0, semG1, semG2, semG3, semG4)

    _zero_aggr(aggr_u, izeros)

    pltpu.sync_copy(cnt_hbm.at[pl.ds(wid * L, L)], cbuf)
    total = jnp.max(cbuf[pl.ds(0, L)])
    nb = (total + (K - 1)) // K

    def _fire_list(b, r):
        o = pl.multiple_of(lbase0 + b * K, 8)
        pltpu.async_copy(sl_hbm.at[pl.ds(o, K)], sidx[r], semsL[r])
        pltpu.async_copy(ll_hbm.at[pl.ds(o, K)], lidx[r], semsL[r])

    def _wait_list(r):
        pltpu.make_async_copy(sl_hbm.at[pl.ds(lbase0, K)],
                              sidx[r], semsL[r]).wait()
        pltpu.make_async_copy(ll_hbm.at[pl.ds(lbase0, K)],
                              lidx[r], semsL[r]).wait()

    def _fire_rows(r):
        pltpu.async_copy(xp_hbm.at[sidx[r]], rows[r], semsG[r])

    def _wait_rows(r):
        pltpu.make_async_copy(xp_hbm.at[sidx[r]], rows[r], semsG[r]).wait()

    for j in range(4):
        @pl.when(j < nb)
        def _():
            _fire_list(jnp.int32(j), j)
    for j in range(2):
        @pl.when(j < nb)
        def _():
            _wait_list(j)
            _fire_rows(j)

    def _step(ts, _):
        for r in range(R2):
            t = ts * R2 + r

            @pl.when(t + 4 < nb)
            def _():
                _fire_list(t + 4, (r + 4) % R2)

            @pl.when(t + 2 < nb)
            def _():
                _wait_list((r + 2) % R2)
                _fire_rows((r + 2) % R2)

            @pl.when(t < nb)
            def _():
                _wait_rows(r)
                _update_block(aggr_u, rows[r], lidx[r], 0, t * K, total, iota)
        return 0
    lax.fori_loop(0, (nb + (R2 - 1)) // R2, _step, 0)

    pltpu.sync_copy(aggr_u.at[pl.ds(0, NPW)], out_hbm.at[pl.ds(lo, NPW)])


_seg2_kernel = functools.partial(
    pl.kernel,
    out_type=jax.ShapeDtypeStruct((NPAD, DW), jnp.int32),
    mesh=plsc.VectorSubcoreMesh(core_axis_name="c", subcore_axis_name="s"),
    compiler_params=pltpu.CompilerParams(needs_layout_passes=False),
    scratch_types=(
        [pltpu.VMEM((K,), jnp.int32) for _ in range(5)]
        + [pltpu.VMEM((K,), jnp.int32) for _ in range(5)]
        + [pltpu.VMEM((K, DW), jnp.int32) for _ in range(5)]
        + [pltpu.VMEM((NPW + 1, DW), jnp.int32)]
        + [pltpu.VMEM((L,), jnp.int32)]
        + [pltpu.SemaphoreType.DMA for _ in range(10)]
    ),
)(_seg2_body)


def _pack(xp_bf16):
    return lax.bitcast_convert_type(xp_bf16.reshape(N, DW, 2), jnp.int32)


def _unpack(aggr_u):
    aggr = lax.bitcast_convert_type(aggr_u, jnp.bfloat16)
    return aggr.reshape(NPAD, D)[:N].astype(jnp.float32)


@jax.jit
def kernel(x, edge_index, Wp1, bp1, Wl1, bl1, Wr1, Wp2, bp2, Wl2, bl2, Wr2):
    src = edge_index[0]
    dst = edge_index[1]

    xp1 = _proj(x, Wp1.T, bp1.reshape(1, D))
    aggr1_u, sl, ll, cnt = _seg1_kernel(_pack(xp1), src, dst)
    h = _out(_unpack(aggr1_u), Wl1.T, bl1.reshape(1, D), x, Wr1.T, True)

    xp2 = _proj(h, Wp2.T, bp2.reshape(1, D))
    aggr2_u = _seg2_kernel(_pack(xp2), sl, ll, cnt)
    return _out(_unpack(aggr2_u), Wl2.T, bl2.reshape(1, D), h, Wr2.T, False)
